# baseline (speedup 1.0000x reference)
import functools

import jax
import jax.numpy as jnp
from jax.experimental import pallas as pl
from jax.experimental.pallas import tpu as pltpu

_KS0 = 0
_KS1 = 42
_KS2 = _KS0 ^ _KS1 ^ 0x1BD11BDA

_ROT_A = (13, 15, 26, 6)
_ROT_B = (17, 29, 16, 24)


def _rotl(x, r):
    return (x << jnp.uint32(r)) | (x >> jnp.uint32(32 - r))


def _threefry_bits(x1):
    ks = (jnp.uint32(_KS0), jnp.uint32(_KS1), jnp.uint32(_KS2))
    x0 = jnp.full_like(x1, jnp.uint32(_KS0))
    for i in range(5):
        rots = _ROT_A if i % 2 == 0 else _ROT_B
        for r in rots:
            x0 = x0 + x1
            x1 = _rotl(x1, r)
            x1 = x1 ^ x0
        x0 = x0 + ks[(i + 1) % 3]
        x1 = x1 + ks[(i + 2) % 3] + jnp.uint32(i + 1)
    return x0 ^ x1


def _sort_body(labels_ref, out_ref, info_ref, c_ref, w_ref):
    labels = labels_ref[...]
    e = (info_ref[...] - labels) ** 2
    d = (out_ref[...] - labels) ** 2
    rows, lanes = e.shape
    n = rows * lanes

    key = jax.lax.bitcast_convert_type(e, jnp.uint32)
    row_id = jax.lax.broadcasted_iota(jnp.int32, (rows, lanes), 0)
    lane_id = jax.lax.broadcasted_iota(jnp.int32, (rows, lanes), 1)
    idx = row_id * lanes + lane_id

    def exchange(x, s):
        if s < lanes:
            up = jnp.roll(x, -s, axis=1)
            dn = jnp.roll(x, s, axis=1)
            mask = (lane_id & s) == 0
        else:
            rs = s // lanes
            up = jnp.roll(x, -rs, axis=0)
            dn = jnp.roll(x, rs, axis=0)
            mask = (row_id & rs) == 0
        return jnp.where(mask, up, dn)

    k = 2
    while k <= n:
        s = k // 2
        while s >= 1:
            if s < lanes:
                lower = (lane_id & s) == 0
            else:
                lower = (row_id & (s // lanes)) == 0
            if k < lanes:
                asc = (lane_id & k) == 0
            elif k < n:
                asc = (row_id & (k // lanes)) == 0
            else:
                asc = jnp.full((rows, lanes), True)
            key_p = exchange(key, s)
            idx_p = exchange(idx, s)
            d_p = exchange(d, s)
            take_min = asc == lower
            self_first = (key < key_p) | ((key == key_p) & (idx < idx_p))
            keep_self = self_first == take_min
            key = jnp.where(keep_self, key, key_p)
            idx = jnp.where(keep_self, idx, idx_p)
            d = jnp.where(keep_self, d, d_p)
            s //= 2
        k *= 2

    c_ref[...] = jnp.log((idx + 1).astype(jnp.float32))
    w_ref[...] = d


def _gumbel_body(c_ref, w_ref, loss_ref, acc_y_ref, acc_w_ref, *,
                 rows_per_step, chunk, bs):
    step = pl.program_id(0)
    n_steps = pl.num_programs(0)
    n_chunks = bs // chunk
    row0 = step * rows_per_step

    row_iota = jax.lax.broadcasted_iota(jnp.int32, (rows_per_step, chunk), 0)
    col_iota = jax.lax.broadcasted_iota(jnp.int32, (rows_per_step, chunk), 1)
    tiny = jnp.float32(jnp.finfo(jnp.float32).tiny)
    x1_base = (row0 + row_iota) * bs + col_iota + _KS1

    acc_y_ref[...] = jnp.full((rows_per_step, chunk), -jnp.inf, jnp.float32)
    acc_w_ref[...] = jnp.zeros((rows_per_step, chunk), jnp.float32)

    def chunk_step(t, _):
        bits = _threefry_bits((x1_base + t * chunk).astype(jnp.uint32))
        fb = (bits >> jnp.uint32(9)) | jnp.uint32(0x3F800000)
        f = jax.lax.bitcast_convert_type(fb, jnp.float32) - jnp.float32(1.0)
        u = f + tiny
        g = -jnp.log(-jnp.log(u))
        y = g + c_ref[pl.ds(t, 1), :]
        acc_y = acc_y_ref[...]
        upd = y > acc_y
        wv = jnp.broadcast_to(w_ref[pl.ds(t, 1), :], y.shape)
        acc_y_ref[...] = jnp.where(upd, y, acc_y)
        acc_w_ref[...] = jnp.where(upd, wv, acc_w_ref[...])
        return 0

    jax.lax.fori_loop(0, n_chunks, chunk_step, 0)
    acc_y = acc_y_ref[...]
    acc_w = acc_w_ref[...]

    m = jnp.max(acc_y, axis=1, keepdims=True)
    pay = jnp.max(jnp.where(acc_y == m, acc_w, jnp.float32(-1.0)), axis=1)
    part = jnp.sum(pay).reshape(1, 1)

    @pl.when(step == 0)
    def _():
        loss_ref[...] = jnp.zeros((1, 1), jnp.float32)

    loss_ref[...] += part

    @pl.when(step == n_steps - 1)
    def _():
        loss_ref[...] = loss_ref[...] / jnp.float32(bs)


@jax.jit
def kernel(i, labels, out, info):
    del i
    bs = labels.shape[0]
    lanes = 128
    rows = bs // lanes
    shape2d = (rows, lanes)

    c, w = pl.pallas_call(
        _sort_body,
        out_shape=(
            jax.ShapeDtypeStruct(shape2d, jnp.float32),
            jax.ShapeDtypeStruct(shape2d, jnp.float32),
        ),
    )(labels.reshape(shape2d), out.reshape(shape2d), info.reshape(shape2d))

    chunk = min(1024, bs)
    n_chunks = bs // chunk
    rows_per_step = 8
    grid = (bs // rows_per_step,)

    c = c.reshape(n_chunks, chunk)
    w = w.reshape(n_chunks, chunk)

    loss = pl.pallas_call(
        functools.partial(
            _gumbel_body, rows_per_step=rows_per_step, chunk=chunk, bs=bs),
        grid=grid,
        in_specs=[
            pl.BlockSpec((n_chunks, chunk), lambda s: (0, 0)),
            pl.BlockSpec((n_chunks, chunk), lambda s: (0, 0)),
        ],
        out_specs=pl.BlockSpec((1, 1), lambda s: (0, 0)),
        out_shape=jax.ShapeDtypeStruct((1, 1), jnp.float32),
        scratch_shapes=[
            pltpu.VMEM((rows_per_step, chunk), jnp.float32),
            pltpu.VMEM((rows_per_step, chunk), jnp.float32),
        ],
    )(c, w)

    return loss.reshape(())


# speedup vs baseline: 1.2159x; 1.2159x over previous
import functools

import jax
import jax.numpy as jnp
from jax.experimental import pallas as pl
from jax.experimental.pallas import tpu as pltpu

_KS0 = 0
_KS1 = 42
_KS2 = _KS0 ^ _KS1 ^ 0x1BD11BDA

_ROT_A = (13, 15, 26, 6)
_ROT_B = (17, 29, 16, 24)


def _rotl(x, r):
    return (x << jnp.uint32(r)) | (x >> jnp.uint32(32 - r))


def _threefry_bits(x1):
    ks = (jnp.uint32(_KS0), jnp.uint32(_KS1), jnp.uint32(_KS2))
    x0 = jnp.full_like(x1, jnp.uint32(_KS0))
    for i in range(5):
        rots = _ROT_A if i % 2 == 0 else _ROT_B
        for r in rots:
            x0 = x0 + x1
            x1 = _rotl(x1, r)
            x1 = x1 ^ x0
        x0 = x0 + ks[(i + 1) % 3]
        x1 = x1 + ks[(i + 2) % 3] + jnp.uint32(i + 1)
    return x0 ^ x1


def _sort_body(labels_ref, out_ref, info_ref, c_ref, w_ref):
    labels = labels_ref[...]
    e = (info_ref[...] - labels) ** 2
    d = (out_ref[...] - labels) ** 2
    rows, lanes = e.shape
    n = rows * lanes

    key = jax.lax.bitcast_convert_type(e, jnp.uint32)
    row_id = jax.lax.broadcasted_iota(jnp.int32, (rows, lanes), 0)
    lane_id = jax.lax.broadcasted_iota(jnp.int32, (rows, lanes), 1)
    idx = row_id * lanes + lane_id

    def exchange(x, s):
        if s < lanes:
            up = jnp.roll(x, -s, axis=1)
            dn = jnp.roll(x, s, axis=1)
            mask = (lane_id & s) == 0
        else:
            rs = s // lanes
            up = jnp.roll(x, -rs, axis=0)
            dn = jnp.roll(x, rs, axis=0)
            mask = (row_id & rs) == 0
        return jnp.where(mask, up, dn)

    k = 2
    while k <= n:
        s = k // 2
        while s >= 1:
            if s < lanes:
                lower = (lane_id & s) == 0
            else:
                lower = (row_id & (s // lanes)) == 0
            if k < lanes:
                asc = (lane_id & k) == 0
            elif k < n:
                asc = (row_id & (k // lanes)) == 0
            else:
                asc = jnp.full((rows, lanes), True)
            key_p = exchange(key, s)
            idx_p = exchange(idx, s)
            d_p = exchange(d, s)
            take_min = asc == lower
            self_first = (key < key_p) | ((key == key_p) & (idx < idx_p))
            keep_self = self_first == take_min
            key = jnp.where(keep_self, key, key_p)
            idx = jnp.where(keep_self, idx, idx_p)
            d = jnp.where(keep_self, d, d_p)
            s //= 2
        k *= 2

    c_ref[...] = jnp.log((idx + 1).astype(jnp.float32))
    w_ref[...] = d


def _gumbel_body(c_ref, w_ref, loss_ref, *, rows_per_step, chunk, bs):
    step = pl.program_id(0)
    n_steps = pl.num_programs(0)
    n_chunks = bs // chunk
    row0 = step * rows_per_step

    row_iota = jax.lax.broadcasted_iota(jnp.int32, (rows_per_step, chunk), 0)
    col_iota = jax.lax.broadcasted_iota(jnp.int32, (rows_per_step, chunk), 1)
    tiny = jnp.float32(jnp.finfo(jnp.float32).tiny)
    x1_base = (row0 + row_iota) * bs + col_iota + _KS1

    def chunk_step(t, carry):
        acc_y, acc_w = carry
        bits = _threefry_bits((x1_base + t * chunk).astype(jnp.uint32))
        fb = (bits >> jnp.uint32(9)) | jnp.uint32(0x3F800000)
        f = jax.lax.bitcast_convert_type(fb, jnp.float32) - jnp.float32(1.0)
        u = f + tiny
        g = -jnp.log(-jnp.log(u))
        y = g + c_ref[t]
        wv = w_ref[t]
        upd = y > acc_y
        return jnp.where(upd, y, acc_y), jnp.where(upd, wv, acc_w)

    acc_y = jnp.full((rows_per_step, chunk), -jnp.inf, dtype=jnp.float32)
    acc_w = jnp.zeros((rows_per_step, chunk), dtype=jnp.float32)
    acc_y, acc_w = jax.lax.fori_loop(0, n_chunks, chunk_step, (acc_y, acc_w))

    m = jnp.max(acc_y, axis=1, keepdims=True)
    pay = jnp.max(jnp.where(acc_y == m, acc_w, jnp.float32(-1.0)), axis=1)
    part = jnp.sum(pay).reshape(1, 1)

    @pl.when(step == 0)
    def _():
        loss_ref[...] = jnp.zeros((1, 1), jnp.float32)

    loss_ref[...] += part

    @pl.when(step == n_steps - 1)
    def _():
        loss_ref[...] = loss_ref[...] / jnp.float32(bs)


@jax.jit
def kernel(i, labels, out, info):
    del i
    bs = labels.shape[0]
    lanes = 128
    rows = bs // lanes
    shape2d = (rows, lanes)

    c, w = pl.pallas_call(
        _sort_body,
        out_shape=(
            jax.ShapeDtypeStruct(shape2d, jnp.float32),
            jax.ShapeDtypeStruct(shape2d, jnp.float32),
        ),
    )(labels.reshape(shape2d), out.reshape(shape2d), info.reshape(shape2d))

    chunk = min(1024, bs)
    n_chunks = bs // chunk
    rows_per_step = 8
    grid = (bs // rows_per_step,)

    c = jnp.broadcast_to(
        c.reshape(n_chunks, 1, chunk), (n_chunks, rows_per_step, chunk))
    w = jnp.broadcast_to(
        w.reshape(n_chunks, 1, chunk), (n_chunks, rows_per_step, chunk))

    loss = pl.pallas_call(
        functools.partial(
            _gumbel_body, rows_per_step=rows_per_step, chunk=chunk, bs=bs),
        grid=grid,
        in_specs=[
            pl.BlockSpec(
                (n_chunks, rows_per_step, chunk), lambda s: (0, 0, 0)),
            pl.BlockSpec(
                (n_chunks, rows_per_step, chunk), lambda s: (0, 0, 0)),
        ],
        out_specs=pl.BlockSpec((1, 1), lambda s: (0, 0)),
        out_shape=jax.ShapeDtypeStruct((1, 1), jnp.float32),
    )(c, w)

    return loss.reshape(())


# speedup vs baseline: 1.4479x; 1.1909x over previous
import functools

import jax
import jax.numpy as jnp
from jax.experimental import pallas as pl
from jax.experimental.pallas import tpu as pltpu

_KS0 = 0
_KS1 = 42
_KS2 = _KS0 ^ _KS1 ^ 0x1BD11BDA

_ROT_A = (13, 15, 26, 6)
_ROT_B = (17, 29, 16, 24)


def _rotl(x, r):
    return (x << jnp.uint32(r)) | (x >> jnp.uint32(32 - r))


def _threefry_bits(x1):
    ks = (jnp.uint32(_KS0), jnp.uint32(_KS1), jnp.uint32(_KS2))
    x0 = jnp.full_like(x1, jnp.uint32(_KS0))
    for i in range(5):
        rots = _ROT_A if i % 2 == 0 else _ROT_B
        for r in rots:
            x0 = x0 + x1
            x1 = _rotl(x1, r)
            x1 = x1 ^ x0
        x0 = x0 + ks[(i + 1) % 3]
        x1 = x1 + ks[(i + 2) % 3] + jnp.uint32(i + 1)
    return x0 ^ x1


def _sort_body(labels_ref, out_ref, info_ref, c_ref, w_ref):
    labels = labels_ref[...]
    e = (info_ref[...] - labels) ** 2
    d = (out_ref[...] - labels) ** 2
    rows, lanes = e.shape
    n = rows * lanes

    key = jax.lax.bitcast_convert_type(e, jnp.uint32)
    row_id = jax.lax.broadcasted_iota(jnp.int32, (rows, lanes), 0)
    lane_id = jax.lax.broadcasted_iota(jnp.int32, (rows, lanes), 1)
    idx = row_id * lanes + lane_id

    def exchange(x, s):
        if s < lanes:
            up = jnp.roll(x, -s, axis=1)
            dn = jnp.roll(x, s, axis=1)
            mask = (lane_id & s) == 0
        else:
            rs = s // lanes
            up = jnp.roll(x, -rs, axis=0)
            dn = jnp.roll(x, rs, axis=0)
            mask = (row_id & rs) == 0
        return jnp.where(mask, up, dn)

    k = 2
    while k <= n:
        s = k // 2
        while s >= 1:
            if s < lanes:
                lower = (lane_id & s) == 0
            else:
                lower = (row_id & (s // lanes)) == 0
            if k < lanes:
                asc = (lane_id & k) == 0
            elif k < n:
                asc = (row_id & (k // lanes)) == 0
            else:
                asc = jnp.full((rows, lanes), True)
            key_p = exchange(key, s)
            idx_p = exchange(idx, s)
            d_p = exchange(d, s)
            take_min = asc == lower
            self_first = (key < key_p) | ((key == key_p) & (idx < idx_p))
            keep_self = self_first == take_min
            key = jnp.where(keep_self, key, key_p)
            idx = jnp.where(keep_self, idx, idx_p)
            d = jnp.where(keep_self, d, d_p)
            s //= 2
        k *= 2

    c_ref[...] = jnp.log((idx + 1).astype(jnp.float32))
    w_ref[...] = d


_LN2 = 0.6931471805599453


def _gumbel_body(c_ref, w_ref, ypart_ref, wpart_ref, *, rows_per_step, chunk,
                 bs):
    step = pl.program_id(0)
    n_chunks = bs // chunk
    row0 = step * rows_per_step

    row_iota = jax.lax.broadcasted_iota(jnp.int32, (rows_per_step, chunk), 0)
    col_iota = jax.lax.broadcasted_iota(jnp.int32, (rows_per_step, chunk), 1)
    ln2 = jnp.float32(_LN2)
    x1_base = (row0 + row_iota) * bs + col_iota + _KS1

    def one_chunk(t, acc_y, acc_w):
        bits = _threefry_bits((x1_base + t * chunk).astype(jnp.uint32))
        fb = (bits >> jnp.uint32(9)) | jnp.uint32(0x3F800000)
        u = jax.lax.bitcast_convert_type(fb, jnp.float32) - jnp.float32(1.0)
        nlu = jnp.log2(u) * (-ln2)
        y = c_ref[t] - jnp.log2(nlu) * ln2
        upd = y > acc_y
        return jnp.where(upd, y, acc_y), jnp.where(upd, w_ref[t], acc_w)

    def chunk_pair(t, carry):
        acc_y, acc_w = carry
        acc_y, acc_w = one_chunk(2 * t, acc_y, acc_w)
        return one_chunk(2 * t + 1, acc_y, acc_w)

    acc_y = jnp.full((rows_per_step, chunk), -jnp.inf, dtype=jnp.float32)
    acc_w = jnp.zeros((rows_per_step, chunk), dtype=jnp.float32)
    acc_y, acc_w = jax.lax.fori_loop(0, n_chunks // 2, chunk_pair,
                                     (acc_y, acc_w))
    if n_chunks % 2:
        acc_y, acc_w = one_chunk(n_chunks - 1, acc_y, acc_w)

    fy, fw = acc_y[:, :128], acc_w[:, :128]
    for kblk in range(1, chunk // 128):
        cy = acc_y[:, kblk * 128:(kblk + 1) * 128]
        cw = acc_w[:, kblk * 128:(kblk + 1) * 128]
        upd = cy > fy
        fy = jnp.where(upd, cy, fy)
        fw = jnp.where(upd, cw, fw)
    ypart_ref[...] = fy
    wpart_ref[...] = fw


def _reduce_body(ypart_ref, wpart_ref, loss_ref, *, bs):
    y = ypart_ref[...]
    m = jnp.max(y, axis=1, keepdims=True)
    pay = jnp.max(jnp.where(y == m, wpart_ref[...], jnp.float32(-1.0)),
                  axis=1)
    loss_ref[...] = (jnp.sum(pay) / jnp.float32(bs)).reshape(1, 1)


@jax.jit
def kernel(i, labels, out, info):
    del i
    bs = labels.shape[0]
    lanes = 128
    rows = bs // lanes
    shape2d = (rows, lanes)

    c, w = pl.pallas_call(
        _sort_body,
        out_shape=(
            jax.ShapeDtypeStruct(shape2d, jnp.float32),
            jax.ShapeDtypeStruct(shape2d, jnp.float32),
        ),
    )(labels.reshape(shape2d), out.reshape(shape2d), info.reshape(shape2d))

    chunk = min(1024, bs)
    n_chunks = bs // chunk
    rows_per_step = 8
    grid = (bs // rows_per_step,)

    c = jnp.broadcast_to(
        c.reshape(n_chunks, 1, chunk), (n_chunks, rows_per_step, chunk))
    w = jnp.broadcast_to(
        w.reshape(n_chunks, 1, chunk), (n_chunks, rows_per_step, chunk))

    ypart, wpart = pl.pallas_call(
        functools.partial(
            _gumbel_body, rows_per_step=rows_per_step, chunk=chunk, bs=bs),
        grid=grid,
        in_specs=[
            pl.BlockSpec(
                (n_chunks, rows_per_step, chunk), lambda s: (0, 0, 0)),
            pl.BlockSpec(
                (n_chunks, rows_per_step, chunk), lambda s: (0, 0, 0)),
        ],
        out_specs=(
            pl.BlockSpec((rows_per_step, 128), lambda s: (s, 0)),
            pl.BlockSpec((rows_per_step, 128), lambda s: (s, 0)),
        ),
        out_shape=(
            jax.ShapeDtypeStruct((bs, 128), jnp.float32),
            jax.ShapeDtypeStruct((bs, 128), jnp.float32),
        ),
    )(c, w)

    loss = pl.pallas_call(
        functools.partial(_reduce_body, bs=bs),
        out_shape=jax.ShapeDtypeStruct((1, 1), jnp.float32),
    )(ypart, wpart)

    return loss.reshape(())
